# baseline (speedup 1.0000x reference)
import functools
import jax
import jax.numpy as jnp
from jax import lax
from jax.experimental import pallas as pl
from jax.experimental.pallas import tpu as pltpu
from jax.experimental.pallas import tpu_sc as plsc

B = 16384
K = 20
D = 16
NREL = 16
NC = 2
NS = 16
NW = NC * NS
NB = B // NW
IDX = 128
CH = 64
NCH = NB // CH

_mesh = plsc.VectorSubcoreMesh(core_axis_name="c", subcore_axis_name="s")
_SC_PARAMS = pltpu.CompilerParams(use_tc_tiling_on_sc=False,
                                  needs_layout_passes=False)


def _vrsqrt(x):
    magic = jnp.full((16,), 0x5F3759DF, jnp.int32)
    y = plsc.bitcast(magic - (plsc.bitcast(x, jnp.int32) >> 1), jnp.float32)
    h = 0.5 * x
    for _ in range(3):
        y = y * (1.5 - h * y * y)
    return y


def _lane_iota():
    return lax.broadcasted_iota(jnp.int32, (16,), 0)


def _sc_ugather_body(users_h, uemb_h, u_out, idx_v, u_v, sem):
    wid = lax.axis_index("s") * NC + lax.axis_index("c")
    base = wid * NB
    nj = NB // IDX
    for j in range(nj):
        pltpu.sync_copy(users_h.at[pl.ds(base + j * IDX, IDX)], idx_v.at[j])
    cps = [pltpu.async_copy(uemb_h.at[idx_v.at[j]],
                            u_v.at[pl.ds(j * IDX, IDX)], sem)
           for j in range(nj)]
    for c in cps:
        c.wait()
    pltpu.sync_copy(u_v, u_out.at[pl.ds(base, NB)])


_sc_ugather = pl.kernel(
    _sc_ugather_body,
    out_type=jax.ShapeDtypeStruct((B, D), jnp.float32),
    mesh=_mesh,
    scratch_types=[pltpu.VMEM((NB // IDX, IDX), jnp.int32),
                   pltpu.VMEM((NB, D), jnp.float32),
                   pltpu.SemaphoreType.DMA],
    compiler_params=_SC_PARAMS,
)


_BLK2 = 1024


def _tc_weights_body(u_ref, idsT_ref, neT_ref, rel_ref, urn_ref, wT_ref,
                     neT2_ref):
    neT2_ref[...] = neT_ref[...]
    rel = rel_ref[...]
    rss = jnp.sum(rel * rel, axis=1, keepdims=True)
    rel_rn = rel * jnp.minimum(lax.rsqrt(jnp.maximum(rss, 0.25)), 1.0)
    u = u_ref[...]
    uss = jnp.sum(u * u, axis=1, keepdims=True)
    u_rn = u * jnp.minimum(lax.rsqrt(jnp.maximum(uss, 0.25)), 1.0)
    urn_ref[...] = u_rn
    PT = lax.dot_general(rel_rn, u_rn, (((1,), (1,)), ((), ())),
                         precision=lax.Precision.HIGHEST,
                         preferred_element_type=jnp.float32)
    ids = idsT_ref[...]
    scores = jnp.zeros(ids.shape, jnp.float32)
    for r in range(NREL):
        scores = scores + jnp.where(ids == r, PT[r:r + 1, :], 0.0)
    m = jnp.max(scores, axis=0, keepdims=True)
    e = jnp.exp(scores - m)
    wT_ref[...] = e / jnp.sum(e, axis=0, keepdims=True)


def _tc_weights(u_rows, nrT, neT, rel):
    g = B // _BLK2
    return pl.pallas_call(
        _tc_weights_body,
        grid=(g,),
        in_specs=[pl.BlockSpec((_BLK2, D), lambda i: (i, 0)),
                  pl.BlockSpec((K, _BLK2), lambda i: (0, i)),
                  pl.BlockSpec((K, _BLK2), lambda i: (0, i)),
                  pl.BlockSpec((NREL, D), lambda i: (0, 0))],
        out_specs=[pl.BlockSpec((_BLK2, D), lambda i: (i, 0)),
                   pl.BlockSpec((K, _BLK2), lambda i: (0, i)),
                   pl.BlockSpec((K, _BLK2), lambda i: (0, i))],
        out_shape=[jax.ShapeDtypeStruct((B, D), jnp.float32),
                   jax.ShapeDtypeStruct((K, B), jnp.float32),
                   jax.ShapeDtypeStruct((K, B), jnp.int32)],
    )(u_rows, nrT, neT, rel)


def _sc_agg_body(items_h, neT_h, wT_h, eemb_h, nv_out, it_out,
                 iidx_v, it_v, ids_v, w_v, rows0_v, rows1_v, nv_v,
                 sem_i, sem0, sem1):
    wid = lax.axis_index("s") * NC + lax.axis_index("c")
    base = wid * NB
    nj = NB // IDX
    for j in range(nj):
        pltpu.sync_copy(items_h.at[pl.ds(base + j * IDX, IDX)], iidx_v.at[j])
    itcps = [pltpu.async_copy(eemb_h.at[iidx_v.at[j]],
                              it_v.at[pl.ds(j * IDX, IDX)], sem_i)
             for j in range(nj)]
    for k in range(K):
        pltpu.sync_copy(neT_h.at[k, pl.ds(base, NB)], ids_v.at[k])
        pltpu.sync_copy(wT_h.at[k, pl.ds(base, NB)], w_v.at[k])

    row_bufs = [rows0_v, rows1_v]
    sems = [sem0, sem1]
    lanes = _lane_iota()

    def start(c):
        return [pltpu.async_copy(
                    eemb_h.at[ids_v.at[k, pl.ds(c * CH, CH)]],
                    row_bufs[c % 2].at[pl.ds(k * CH, CH)],
                    sems[c % 2])
                for k in range(K)]

    pending = start(0)

    for c in range(NCH):
        for cp in pending:
            cp.wait()
        if c + 1 < NCH:
            pending = start(c + 1)
        buf = row_bufs[c % 2]

        def g_body(g, carry, buf=buf, c=c):
            i0 = g * 16
            acc_a = [jnp.zeros((16,), jnp.float32) for _ in range(D)]
            for k in range(K):
                ridx = k * CH + i0 + lanes
                cols = [plsc.load_gather(buf, [ridx, jnp.full((16,), d, jnp.int32)])
                        for d in range(D)]
                sq = [cols[d] * cols[d] for d in range(D)]
                for step in (1, 2, 4, 8):
                    for d in range(0, D, 2 * step):
                        sq[d] = sq[d] + sq[d + step]
                scale = jnp.minimum(_vrsqrt(jnp.maximum(sq[0], 0.25)), 1.0)
                wk = w_v[k, pl.ds(c * CH + i0, 16)]
                cc = wk * scale
                for d in range(D):
                    acc_a[d] = acc_a[d] + cc * cols[d]
            item = c * CH + i0 + lanes
            for d in range(D):
                plsc.store_scatter(nv_v, [item, jnp.full((16,), d, jnp.int32)],
                                   acc_a[d])
            return carry

        lax.fori_loop(0, CH // 16, g_body, 0)

    for cp in itcps:
        cp.wait()
    pltpu.sync_copy(it_v, it_out.at[pl.ds(base, NB)])
    pltpu.sync_copy(nv_v, nv_out.at[pl.ds(base, NB)])


_sc_agg = pl.kernel(
    _sc_agg_body,
    out_type=[jax.ShapeDtypeStruct((B, D), jnp.float32),
              jax.ShapeDtypeStruct((B, D), jnp.float32)],
    mesh=_mesh,
    scratch_types=[pltpu.VMEM((NB // IDX, IDX), jnp.int32),
                   pltpu.VMEM((NB, D), jnp.float32),
                   pltpu.VMEM((K, NB), jnp.int32),
                   pltpu.VMEM((K, NB), jnp.float32),
                   pltpu.VMEM((CH * K, D), jnp.float32),
                   pltpu.VMEM((CH * K, D), jnp.float32),
                   pltpu.VMEM((NB, D), jnp.float32),
                   pltpu.SemaphoreType.DMA,
                   pltpu.SemaphoreType.DMA,
                   pltpu.SemaphoreType.DMA],
    compiler_params=_SC_PARAMS,
)


_BLK4 = 2048


def _tc_final_body(it_ref, nv_ref, urn_ref, W_ref, b_ref, y_ref):
    it = it_ref[...]
    iss = jnp.sum(it * it, axis=1, keepdims=True)
    it_rn = it * jnp.minimum(lax.rsqrt(jnp.maximum(iss, 0.25)), 1.0)
    h = it_rn + nv_ref[...]
    o = lax.dot_general(h, W_ref[...], (((1,), (1,)), ((), ())),
                        precision=lax.Precision.HIGHEST,
                        preferred_element_type=jnp.float32)
    o = jnp.maximum(o + b_ref[...], 0.0)
    t = jnp.sum(urn_ref[...] * o, axis=1)
    y_ref[...] = 1.0 / (1.0 + jnp.exp(-t))


def _tc_final(it_rows, nv, u_rn, W, b2):
    g = B // _BLK4
    return pl.pallas_call(
        _tc_final_body,
        grid=(g,),
        in_specs=[pl.BlockSpec((_BLK4, D), lambda i: (i, 0)),
                  pl.BlockSpec((_BLK4, D), lambda i: (i, 0)),
                  pl.BlockSpec((_BLK4, D), lambda i: (i, 0)),
                  pl.BlockSpec((D, D), lambda i: (0, 0)),
                  pl.BlockSpec((1, D), lambda i: (0, 0))],
        out_specs=pl.BlockSpec((_BLK4,), lambda i: (i,)),
        out_shape=jax.ShapeDtypeStruct((B,), jnp.float32),
    )(it_rows, nv, u_rn, W, b2)


def kernel(users, items, adj_entity, adj_relation, user_emb, entity_emb,
           relation_emb, W, b):
    users = users.astype(jnp.int32)
    items = items.astype(jnp.int32)
    neT = jnp.take(adj_entity, items, axis=0).T.astype(jnp.int32)
    nrT = jnp.take(adj_relation, items, axis=0).T.astype(jnp.int32)
    u_rows = _sc_ugather(users, user_emb)
    u_rn, wT, neT2 = _tc_weights(u_rows, nrT, neT, relation_emb)
    nv, it_rows = _sc_agg(items, neT2, wT, entity_emb)
    return _tc_final(it_rows, nv, u_rn, W, b.reshape(1, D))


# speedup vs baseline: 1.0297x; 1.0297x over previous
import functools
import jax
import jax.numpy as jnp
from jax import lax
from jax.experimental import pallas as pl
from jax.experimental.pallas import tpu as pltpu
from jax.experimental.pallas import tpu_sc as plsc

B = 16384
K = 20
D = 16
NREL = 16
NC = 2
NS = 16
NW = NC * NS
NB = B // NW
IDX = 128
CH = 64
NCH = NB // CH

_mesh = plsc.VectorSubcoreMesh(core_axis_name="c", subcore_axis_name="s")
_SC_PARAMS = pltpu.CompilerParams(use_tc_tiling_on_sc=False,
                                  needs_layout_passes=False)


def _vrsqrt(x):
    magic = jnp.full((16,), 0x5F3759DF, jnp.int32)
    y = plsc.bitcast(magic - (plsc.bitcast(x, jnp.int32) >> 1), jnp.float32)
    h = 0.5 * x
    for _ in range(3):
        y = y * (1.5 - h * y * y)
    return y


def _lane_iota():
    return lax.broadcasted_iota(jnp.int32, (16,), 0)


def _sc_ugather_body(users_h, uemb_h, u_out, idx_v, u_v, sem):
    wid = lax.axis_index("s") * NC + lax.axis_index("c")
    base = wid * NB
    nj = NB // IDX
    for j in range(nj):
        pltpu.sync_copy(users_h.at[pl.ds(base + j * IDX, IDX)], idx_v.at[j])
    cps = [pltpu.async_copy(uemb_h.at[idx_v.at[j]],
                            u_v.at[pl.ds(j * IDX, IDX)], sem)
           for j in range(nj)]
    for c in cps:
        c.wait()
    pltpu.sync_copy(u_v, u_out.at[pl.ds(base, NB)])


_sc_ugather = pl.kernel(
    _sc_ugather_body,
    out_type=jax.ShapeDtypeStruct((B, D), jnp.float32),
    mesh=_mesh,
    scratch_types=[pltpu.VMEM((NB // IDX, IDX), jnp.int32),
                   pltpu.VMEM((NB, D), jnp.float32),
                   pltpu.SemaphoreType.DMA],
    compiler_params=_SC_PARAMS,
)


_BLK2 = 1024


def _tc_weights_body(u_ref, idsT_ref, neT_ref, rel_ref, urn_ref, wT_ref,
                     neT2_ref):
    neT2_ref[...] = neT_ref[...]
    rel = rel_ref[...]
    rss = jnp.sum(rel * rel, axis=1, keepdims=True)
    rel_rn = rel * jnp.minimum(lax.rsqrt(jnp.maximum(rss, 0.25)), 1.0)
    u = u_ref[...]
    uss = jnp.sum(u * u, axis=1, keepdims=True)
    u_rn = u * jnp.minimum(lax.rsqrt(jnp.maximum(uss, 0.25)), 1.0)
    urn_ref[...] = u_rn
    PT = lax.dot_general(rel_rn, u_rn, (((1,), (1,)), ((), ())),
                         precision=lax.Precision.HIGHEST,
                         preferred_element_type=jnp.float32)
    ids = idsT_ref[...]
    scores = jnp.zeros(ids.shape, jnp.float32)
    for r in range(NREL):
        scores = scores + jnp.where(ids == r, PT[r:r + 1, :], 0.0)
    m = jnp.max(scores, axis=0, keepdims=True)
    e = jnp.exp(scores - m)
    wT_ref[...] = e / jnp.sum(e, axis=0, keepdims=True)


def _tc_weights(u_rows, nrT, neT, rel):
    g = B // _BLK2
    return pl.pallas_call(
        _tc_weights_body,
        grid=(g,),
        in_specs=[pl.BlockSpec((_BLK2, D), lambda i: (i, 0)),
                  pl.BlockSpec((K, _BLK2), lambda i: (0, i)),
                  pl.BlockSpec((K, _BLK2), lambda i: (0, i)),
                  pl.BlockSpec((NREL, D), lambda i: (0, 0))],
        out_specs=[pl.BlockSpec((_BLK2, D), lambda i: (i, 0)),
                   pl.BlockSpec((K, _BLK2), lambda i: (0, i)),
                   pl.BlockSpec((K, _BLK2), lambda i: (0, i))],
        out_shape=[jax.ShapeDtypeStruct((B, D), jnp.float32),
                   jax.ShapeDtypeStruct((K, B), jnp.float32),
                   jax.ShapeDtypeStruct((K, B), jnp.int32)],
    )(u_rows, nrT, neT, rel)


def _sc_agg_body(items_h, neT_h, wT_h, eemb_h, nv_out, it_out,
                 iidx_v, it_v, ids_v, w_v, rows0_v, rows1_v, nv_v,
                 sem_i, sem0, sem1):
    wid = lax.axis_index("s") * NC + lax.axis_index("c")
    base = wid * NB
    nj = NB // IDX
    for j in range(nj):
        pltpu.sync_copy(items_h.at[pl.ds(base + j * IDX, IDX)], iidx_v.at[j])
    itcps = [pltpu.async_copy(eemb_h.at[iidx_v.at[j]],
                              it_v.at[pl.ds(j * IDX, IDX)], sem_i)
             for j in range(nj)]
    stcps = []
    for k in range(K):
        stcps.append(pltpu.async_copy(neT_h.at[k, pl.ds(base, NB)],
                                      ids_v.at[k], sem_i))
        stcps.append(pltpu.async_copy(wT_h.at[k, pl.ds(base, NB)],
                                      w_v.at[k], sem_i))
    for cp in stcps:
        cp.wait()

    row_bufs = [rows0_v, rows1_v]
    sems = [sem0, sem1]
    lanes = _lane_iota()

    def start(c):
        return [pltpu.async_copy(
                    eemb_h.at[ids_v.at[k, pl.ds(c * CH, CH)]],
                    row_bufs[c % 2].at[pl.ds(k * CH, CH)],
                    sems[c % 2])
                for k in range(K)]

    pending = start(0)

    for c in range(NCH):
        for cp in pending:
            cp.wait()
        if c + 1 < NCH:
            pending = start(c + 1)
        buf = row_bufs[c % 2]

        def g_body(g, carry, buf=buf, c=c):
            i0 = g * 16
            acc_a = [jnp.zeros((16,), jnp.float32) for _ in range(D)]
            for k in range(K):
                ridx = k * CH + i0 + lanes
                cols = [plsc.load_gather(buf, [ridx, jnp.full((16,), d, jnp.int32)])
                        for d in range(D)]
                sq = [cols[d] * cols[d] for d in range(D)]
                for step in (1, 2, 4, 8):
                    for d in range(0, D, 2 * step):
                        sq[d] = sq[d] + sq[d + step]
                scale = jnp.minimum(_vrsqrt(jnp.maximum(sq[0], 0.25)), 1.0)
                wk = w_v[k, pl.ds(c * CH + i0, 16)]
                cc = wk * scale
                for d in range(D):
                    acc_a[d] = acc_a[d] + cc * cols[d]
            item = c * CH + i0 + lanes
            for d in range(D):
                plsc.store_scatter(nv_v, [item, jnp.full((16,), d, jnp.int32)],
                                   acc_a[d])
            return carry

        lax.fori_loop(0, CH // 16, g_body, 0)

    for cp in itcps:
        cp.wait()
    pltpu.sync_copy(it_v, it_out.at[pl.ds(base, NB)])
    pltpu.sync_copy(nv_v, nv_out.at[pl.ds(base, NB)])


_sc_agg = pl.kernel(
    _sc_agg_body,
    out_type=[jax.ShapeDtypeStruct((B, D), jnp.float32),
              jax.ShapeDtypeStruct((B, D), jnp.float32)],
    mesh=_mesh,
    scratch_types=[pltpu.VMEM((NB // IDX, IDX), jnp.int32),
                   pltpu.VMEM((NB, D), jnp.float32),
                   pltpu.VMEM((K, NB), jnp.int32),
                   pltpu.VMEM((K, NB), jnp.float32),
                   pltpu.VMEM((CH * K, D), jnp.float32),
                   pltpu.VMEM((CH * K, D), jnp.float32),
                   pltpu.VMEM((NB, D), jnp.float32),
                   pltpu.SemaphoreType.DMA,
                   pltpu.SemaphoreType.DMA,
                   pltpu.SemaphoreType.DMA],
    compiler_params=_SC_PARAMS,
)


_BLK4 = 2048


def _tc_final_body(it_ref, nv_ref, urn_ref, W_ref, b_ref, y_ref):
    it = it_ref[...]
    iss = jnp.sum(it * it, axis=1, keepdims=True)
    it_rn = it * jnp.minimum(lax.rsqrt(jnp.maximum(iss, 0.25)), 1.0)
    h = it_rn + nv_ref[...]
    o = lax.dot_general(h, W_ref[...], (((1,), (1,)), ((), ())),
                        precision=lax.Precision.HIGHEST,
                        preferred_element_type=jnp.float32)
    o = jnp.maximum(o + b_ref[...], 0.0)
    t = jnp.sum(urn_ref[...] * o, axis=1)
    y_ref[...] = 1.0 / (1.0 + jnp.exp(-t))


def _tc_final(it_rows, nv, u_rn, W, b2):
    g = B // _BLK4
    return pl.pallas_call(
        _tc_final_body,
        grid=(g,),
        in_specs=[pl.BlockSpec((_BLK4, D), lambda i: (i, 0)),
                  pl.BlockSpec((_BLK4, D), lambda i: (i, 0)),
                  pl.BlockSpec((_BLK4, D), lambda i: (i, 0)),
                  pl.BlockSpec((D, D), lambda i: (0, 0)),
                  pl.BlockSpec((1, D), lambda i: (0, 0))],
        out_specs=pl.BlockSpec((_BLK4,), lambda i: (i,)),
        out_shape=jax.ShapeDtypeStruct((B,), jnp.float32),
    )(it_rows, nv, u_rn, W, b2)


def kernel(users, items, adj_entity, adj_relation, user_emb, entity_emb,
           relation_emb, W, b):
    users = users.astype(jnp.int32)
    items = items.astype(jnp.int32)
    neT = jnp.take(adj_entity, items, axis=0).T.astype(jnp.int32)
    nrT = jnp.take(adj_relation, items, axis=0).T.astype(jnp.int32)
    u_rows = _sc_ugather(users, user_emb)
    u_rn, wT, neT2 = _tc_weights(u_rows, nrT, neT, relation_emb)
    nv, it_rows = _sc_agg(items, neT2, wT, entity_emb)
    return _tc_final(it_rows, nv, u_rn, W, b.reshape(1, D))


# speedup vs baseline: 1.0299x; 1.0003x over previous
import jax
import jax.numpy as jnp
from jax import lax
from jax.experimental import pallas as pl
from jax.experimental.pallas import tpu as pltpu
from jax.experimental.pallas import tpu_sc as plsc

B = 16384
K = 20
D = 16
NREL = 16
NC = 2
NS = 16
NW = NC * NS
NB = B // NW
IDX = 128
CH = 64
NCH = NB // CH

_mesh = plsc.VectorSubcoreMesh(core_axis_name="c", subcore_axis_name="s")
_SC_PARAMS = pltpu.CompilerParams(use_tc_tiling_on_sc=False,
                                  needs_layout_passes=False)


def _vrsqrt(x):
    magic = jnp.full((16,), 0x5F3759DF, jnp.int32)
    y = plsc.bitcast(magic - (plsc.bitcast(x, jnp.int32) >> 1), jnp.float32)
    h = 0.5 * x
    for _ in range(3):
        y = y * (1.5 - h * y * y)
    return y


def _lane_iota():
    return lax.broadcasted_iota(jnp.int32, (16,), 0)


def _sc_ugather_body(users_h, uemb_h, u_out, idx_v, u_v, sem):
    wid = lax.axis_index("s") * NC + lax.axis_index("c")
    base = wid * NB
    nj = NB // IDX
    for j in range(nj):
        pltpu.sync_copy(users_h.at[pl.ds(base + j * IDX, IDX)], idx_v.at[j])
    cps = [pltpu.async_copy(uemb_h.at[idx_v.at[j]],
                            u_v.at[pl.ds(j * IDX, IDX)], sem)
           for j in range(nj)]
    for c in cps:
        c.wait()
    pltpu.sync_copy(u_v, u_out.at[pl.ds(base, NB)])


_sc_ugather = pl.kernel(
    _sc_ugather_body,
    out_type=jax.ShapeDtypeStruct((B, D), jnp.float32),
    mesh=_mesh,
    scratch_types=[pltpu.VMEM((NB // IDX, IDX), jnp.int32),
                   pltpu.VMEM((NB, D), jnp.float32),
                   pltpu.SemaphoreType.DMA],
    compiler_params=_SC_PARAMS,
)


_BLK2 = 1024


def _tc_weights_body(u_ref, idsT_ref, neT_ref, rel_ref, urn_ref, wT_ref,
                     neT2_ref):
    neT2_ref[...] = neT_ref[...]
    rel = rel_ref[...]
    rss = jnp.sum(rel * rel, axis=1, keepdims=True)
    rel_rn = rel * jnp.minimum(lax.rsqrt(jnp.maximum(rss, 0.25)), 1.0)
    u = u_ref[...]
    uss = jnp.sum(u * u, axis=1, keepdims=True)
    u_rn = u * jnp.minimum(lax.rsqrt(jnp.maximum(uss, 0.25)), 1.0)
    urn_ref[...] = u_rn
    PT = lax.dot_general(rel_rn, u_rn, (((1,), (1,)), ((), ())),
                         precision=lax.Precision.HIGHEST,
                         preferred_element_type=jnp.float32)
    ids = idsT_ref[...]
    scores = jnp.zeros(ids.shape, jnp.float32)
    for r in range(NREL):
        scores = scores + jnp.where(ids == r, PT[r:r + 1, :], 0.0)
    m = jnp.max(scores, axis=0, keepdims=True)
    e = jnp.exp(scores - m)
    wT_ref[...] = e / jnp.sum(e, axis=0, keepdims=True)


def _tc_weights(u_rows, nrT, neT, rel):
    g = B // _BLK2
    return pl.pallas_call(
        _tc_weights_body,
        grid=(g,),
        in_specs=[pl.BlockSpec((_BLK2, D), lambda i: (i, 0)),
                  pl.BlockSpec((K, _BLK2), lambda i: (0, i)),
                  pl.BlockSpec((K, _BLK2), lambda i: (0, i)),
                  pl.BlockSpec((NREL, D), lambda i: (0, 0))],
        out_specs=[pl.BlockSpec((_BLK2, D), lambda i: (i, 0)),
                   pl.BlockSpec((K, _BLK2), lambda i: (0, i)),
                   pl.BlockSpec((K, _BLK2), lambda i: (0, i))],
        out_shape=[jax.ShapeDtypeStruct((B, D), jnp.float32),
                   jax.ShapeDtypeStruct((K, B), jnp.float32),
                   jax.ShapeDtypeStruct((K, B), jnp.int32)],
    )(u_rows, nrT, neT, rel)


def _sc_agg_body(items_h, neT_h, wT_h, eemb_h, nv_out, it_out,
                 iidx_v, it_v, ids_v, w_v, rows0_v, rows1_v, nv_v,
                 sem_i, sem0, sem1):
    wid = lax.axis_index("s") * NC + lax.axis_index("c")
    base = wid * NB
    nj = NB // IDX
    for j in range(nj):
        pltpu.sync_copy(items_h.at[pl.ds(base + j * IDX, IDX)], iidx_v.at[j])
    itcps = [pltpu.async_copy(eemb_h.at[iidx_v.at[j]],
                              it_v.at[pl.ds(j * IDX, IDX)], sem_i)
             for j in range(nj)]
    stcps = []
    for k in range(K):
        stcps.append(pltpu.async_copy(neT_h.at[k, pl.ds(base, NB)],
                                      ids_v.at[k], sem_i))
        stcps.append(pltpu.async_copy(wT_h.at[k, pl.ds(base, NB)],
                                      w_v.at[k], sem_i))
    for cp in stcps:
        cp.wait()

    row_bufs = [rows0_v, rows1_v]
    sems = [sem0, sem1]
    lanes = _lane_iota()

    def start(c):
        return [pltpu.async_copy(
                    eemb_h.at[ids_v.at[k, pl.ds(c * CH, CH)]],
                    row_bufs[c % 2].at[pl.ds(k * CH, CH)],
                    sems[c % 2])
                for k in range(K)]

    pending = start(0)

    for c in range(NCH):
        for cp in pending:
            cp.wait()
        if c + 1 < NCH:
            pending = start(c + 1)
        buf = row_bufs[c % 2]

        def g_body(g, carry, buf=buf, c=c):
            i0 = g * 16
            acc_a = [jnp.zeros((16,), jnp.float32) for _ in range(D)]
            for k in range(K):
                ridx = k * CH + i0 + lanes
                cols = [plsc.load_gather(buf, [ridx, jnp.full((16,), d, jnp.int32)])
                        for d in range(D)]
                sq = [cols[d] * cols[d] for d in range(D)]
                for step in (1, 2, 4, 8):
                    for d in range(0, D, 2 * step):
                        sq[d] = sq[d] + sq[d + step]
                scale = jnp.minimum(_vrsqrt(jnp.maximum(sq[0], 0.25)), 1.0)
                wk = w_v[k, pl.ds(c * CH + i0, 16)]
                cc = wk * scale
                for d in range(D):
                    acc_a[d] = acc_a[d] + cc * cols[d]
            item = c * CH + i0 + lanes
            for d in range(D):
                plsc.store_scatter(nv_v, [item, jnp.full((16,), d, jnp.int32)],
                                   acc_a[d])
            return carry

        lax.fori_loop(0, CH // 16, g_body, 0)

    for cp in itcps:
        cp.wait()
    pltpu.sync_copy(it_v, it_out.at[pl.ds(base, NB)])
    pltpu.sync_copy(nv_v, nv_out.at[pl.ds(base, NB)])


_sc_agg = pl.kernel(
    _sc_agg_body,
    out_type=[jax.ShapeDtypeStruct((B, D), jnp.float32),
              jax.ShapeDtypeStruct((B, D), jnp.float32)],
    mesh=_mesh,
    scratch_types=[pltpu.VMEM((NB // IDX, IDX), jnp.int32),
                   pltpu.VMEM((NB, D), jnp.float32),
                   pltpu.VMEM((K, NB), jnp.int32),
                   pltpu.VMEM((K, NB), jnp.float32),
                   pltpu.VMEM((CH * K, D), jnp.float32),
                   pltpu.VMEM((CH * K, D), jnp.float32),
                   pltpu.VMEM((NB, D), jnp.float32),
                   pltpu.SemaphoreType.DMA,
                   pltpu.SemaphoreType.DMA,
                   pltpu.SemaphoreType.DMA],
    compiler_params=_SC_PARAMS,
)


_BLK4 = 2048


def _tc_final_body(it_ref, nv_ref, urn_ref, W_ref, b_ref, y_ref):
    it = it_ref[...]
    iss = jnp.sum(it * it, axis=1, keepdims=True)
    it_rn = it * jnp.minimum(lax.rsqrt(jnp.maximum(iss, 0.25)), 1.0)
    h = it_rn + nv_ref[...]
    o = lax.dot_general(h, W_ref[...], (((1,), (1,)), ((), ())),
                        precision=lax.Precision.HIGHEST,
                        preferred_element_type=jnp.float32)
    o = jnp.maximum(o + b_ref[...], 0.0)
    t = jnp.sum(urn_ref[...] * o, axis=1)
    y_ref[...] = 1.0 / (1.0 + jnp.exp(-t))


def _tc_final(it_rows, nv, u_rn, W, b2):
    g = B // _BLK4
    return pl.pallas_call(
        _tc_final_body,
        grid=(g,),
        in_specs=[pl.BlockSpec((_BLK4, D), lambda i: (i, 0)),
                  pl.BlockSpec((_BLK4, D), lambda i: (i, 0)),
                  pl.BlockSpec((_BLK4, D), lambda i: (i, 0)),
                  pl.BlockSpec((D, D), lambda i: (0, 0)),
                  pl.BlockSpec((1, D), lambda i: (0, 0))],
        out_specs=pl.BlockSpec((_BLK4,), lambda i: (i,)),
        out_shape=jax.ShapeDtypeStruct((B,), jnp.float32),
    )(it_rows, nv, u_rn, W, b2)


def kernel(users, items, adj_entity, adj_relation, user_emb, entity_emb,
           relation_emb, W, b):
    users = users.astype(jnp.int32)
    items = items.astype(jnp.int32)
    neT = jnp.take(adj_entity, items, axis=0).T.astype(jnp.int32)
    nrT = jnp.take(adj_relation, items, axis=0).T.astype(jnp.int32)
    u_rows = _sc_ugather(users, user_emb)
    u_rn, wT, neT2 = _tc_weights(u_rows, nrT, neT, relation_emb)
    nv, it_rows = _sc_agg(items, neT2, wT, entity_emb)
    return _tc_final(it_rows, nv, u_rn, W, b.reshape(1, D))
